# baseline (speedup 1.0000x reference)
import functools

import jax
import jax.numpy as jnp
from jax import lax
from jax.experimental import pallas as pl
from jax.experimental.pallas import tpu as pltpu
from jax.experimental.pallas import tpu_sc as plsc

BT = 256
T = 256
NWORK = 32
LANES = 16


def _k1_body(E, x_ref, W1_ref, b1_ref, W2_ref, b2_ref, Wg_ref,
             h_ref, i1_ref, i2_ref, w1_ref, w2_ref, h1c_ref, h2c_ref):
    EP = Wg_ref.shape[1]
    bf = jnp.bfloat16
    x = x_ref[...]
    h = jnp.maximum(jnp.dot(x.astype(bf), W1_ref[...].astype(bf),
                            preferred_element_type=jnp.float32)
                    + b1_ref[...], 0.0)
    h = jnp.maximum(jnp.dot(h.astype(bf), W2_ref[...].astype(bf),
                            preferred_element_type=jnp.float32)
                    + b2_ref[...], 0.0)
    h_ref[...] = h.astype(bf)
    logits = jnp.dot(h, Wg_ref[...], preferred_element_type=jnp.float32)
    col = lax.broadcasted_iota(jnp.int32, (BT, EP), 1)
    logits = jnp.where(col < E, logits, -1e30)
    m = jnp.max(logits, axis=1, keepdims=True)
    ex = jnp.exp(logits - m)
    gates = ex / jnp.sum(ex, axis=1, keepdims=True)
    m1 = jnp.max(gates, axis=1, keepdims=True)
    i1 = jnp.min(jnp.where(gates == m1, col, EP), axis=1, keepdims=True)
    g2 = jnp.where(col == i1, -1.0, gates)
    m2 = jnp.max(g2, axis=1, keepdims=True)
    i2 = jnp.min(jnp.where(g2 == m2, col, EP), axis=1, keepdims=True)
    denom = m1 + m2 + 1e-9
    i1_ref[...] = i1
    i2_ref[...] = i2
    w1_ref[...] = m1 / denom
    w2_ref[...] = m2 / denom
    oh1 = jnp.where(col == i1, 1.0, 0.0)
    oh2 = jnp.where(col == i2, 1.0, 0.0)
    half = BT // 2
    h1c_ref[...] = jnp.concatenate(
        [jnp.sum(oh1[:half], axis=0, keepdims=True),
         jnp.sum(oh1[half:], axis=0, keepdims=True)], axis=0)[None]
    h2c_ref[...] = jnp.concatenate(
        [jnp.sum(oh2[:half], axis=0, keepdims=True),
         jnp.sum(oh2[half:], axis=0, keepdims=True)], axis=0)[None]


def _gather16(v, idx):
    dnums = lax.GatherDimensionNumbers(
        offset_dims=(), collapsed_slice_dims=(0,), start_index_map=(0,))
    return lax.gather(v, idx[:, None], dnums, slice_sizes=(1,),
                      mode=lax.GatherScatterMode.PROMISE_IN_BOUNDS)


def _splat16(v, j):
    return _gather16(v, jnp.full((LANES,), j, jnp.int32))


def _prefix16(v, lane):
    for s in (1, 2, 4, 8):
        sh = _gather16(v, jnp.maximum(lane - s, 0))
        v = v + jnp.where(lane >= s, sh, jnp.zeros_like(v))
    return v


def _k2_body(E, NP, NBLKP, TPW,
             h_hbm, eid1_hbm, eid2_hbm, hc1_hbm, hc2_hbm,
             hs_hbm, pos1_hbm, pos2_hbm, blk_hbm,
             eid1_v, eid2_v, pos1_v, pos2_v,
             hc1_v, hc2_v, hbuf, blk_v,
             s1, s2):
    wid = lax.axis_index("s") * 2 + lax.axis_index("c")
    tbase = wid * TPW
    lane = lax.broadcasted_iota(jnp.int32, (LANES,), 0)

    pltpu.sync_copy(eid1_hbm.at[pl.ds(tbase, TPW)], eid1_v)
    pltpu.sync_copy(eid2_hbm.at[pl.ds(tbase, TPW)], eid2_v)
    pltpu.sync_copy(hc1_hbm, hc1_v)
    pltpu.sync_copy(hc2_hbm, hc2_v)

    zf = jnp.zeros((LANES,), jnp.float32)
    tot1 = zf
    tot2 = zf
    pre1 = zf
    pre2 = zf
    for w in range(NWORK):
        r1 = hc1_v[w, pl.ds(0, LANES)]
        r2 = hc2_v[w, pl.ds(0, LANES)]
        tot1 = tot1 + r1
        tot2 = tot2 + r2
        sel = w < wid
        pre1 = pre1 + jnp.where(sel, r1, zf)
        pre2 = pre2 + jnp.where(sel, r2, zf)
    tot = (tot1 + tot2).astype(jnp.int32)
    pc = ((tot + (T - 1)) >> 8) << 8
    pcs = [_splat16(pc, e) for e in range(E)]
    ps = [jnp.zeros((LANES,), jnp.int32)]
    for e in range(1, E):
        ps.append(ps[e - 1] + pcs[e - 1])
    tot1i = tot1.astype(jnp.int32)
    pre1i = pre1.astype(jnp.int32)
    pre2i = pre2.astype(jnp.int32)
    cur1 = [ps[e] + _splat16(pre1i, e) for e in range(E)]
    cur2 = [ps[e] + _splat16(tot1i, e) + _splat16(pre2i, e) for e in range(E)]
    ends = [ps[e] + pcs[e] for e in range(E)]

    @pl.when(wid == 0)
    def _():
        for c in range(NBLKP // LANES):
            bstart = (c * LANES + lane) * T
            acc = jnp.zeros((LANES,), jnp.int32)
            for e in range(E):
                acc = acc + jnp.where(ends[e] <= bstart, 1, 0)
            blk_v[pl.ds(c * LANES, LANES)] = jnp.minimum(acc, E - 1)
        pltpu.sync_copy(blk_v, blk_hbm)

    nch = TPW // LANES
    for eid_v, pos_v, cur in ((eid1_v, pos1_v, cur1), (eid2_v, pos2_v, cur2)):
        for c in range(nch):
            ev = eid_v[pl.ds(c * LANES, LANES)]
            posv = jnp.zeros((LANES,), jnp.int32)
            for e in range(E):
                msk = ev == e
                ones = jnp.where(msk, 1, 0)
                cs = _prefix16(ones, lane)
                posv = jnp.where(msk, cur[e] + cs - 1, posv)
                cur[e] = cur[e] + _splat16(cs, LANES - 1)
            pos_v[pl.ds(c * LANES, LANES)] = posv
    pltpu.sync_copy(pos1_v, pos1_hbm.at[pl.ds(tbase, TPW)])
    pltpu.sync_copy(pos2_v, pos2_hbm.at[pl.ds(tbase, TPW)])

    pltpu.sync_copy(h_hbm.at[pl.ds(tbase, TPW)], hbuf)
    c1 = pltpu.async_copy(hbuf, hs_hbm.at[pos1_v], s1)
    c2 = pltpu.async_copy(hbuf, hs_hbm.at[pos2_v], s2)
    c1.wait()
    c2.wait()


def _k3_body(blk_ref, hs_ref, We_ref, be_ref, y_ref):
    bf = jnp.bfloat16
    y = jnp.dot(hs_ref[...], We_ref[0].astype(bf),
                preferred_element_type=jnp.float32) + be_ref[0]
    y_ref[...] = y.astype(bf)


def _k4_body(TPW, y_hbm, pos1_hbm, pos2_hbm, y1_hbm, y2_hbm,
             pos1_v, pos2_v, yb, s1, s2):
    wid = lax.axis_index("s") * 2 + lax.axis_index("c")
    tbase = wid * TPW
    pltpu.sync_copy(pos1_hbm.at[pl.ds(tbase, TPW)], pos1_v)
    pltpu.sync_copy(pos2_hbm.at[pl.ds(tbase, TPW)], pos2_v)
    g1 = pltpu.async_copy(y_hbm.at[pos1_v], yb, s1)
    g1.wait()
    pltpu.sync_copy(yb, y1_hbm.at[pl.ds(tbase, TPW)])
    g2 = pltpu.async_copy(y_hbm.at[pos2_v], yb, s2)
    g2.wait()
    pltpu.sync_copy(yb, y2_hbm.at[pl.ds(tbase, TPW)])


def _k5_body(NC, y1_ref, y2_ref, w1_ref, w2_ref, Wp_ref, bp_ref, out_ref):
    bf = jnp.bfloat16
    NCP = Wp_ref.shape[1]
    hm = (w1_ref[...] * y1_ref[...].astype(jnp.float32)
          + w2_ref[...] * y2_ref[...].astype(jnp.float32))
    lg = jnp.dot(hm.astype(bf), Wp_ref[...].astype(bf),
                 preferred_element_type=jnp.float32) + bp_ref[...]
    colc = lax.broadcasted_iota(jnp.int32, (BT, NCP), 1)
    lg = jnp.where(colc < NC, lg, -1e30)
    mm = jnp.max(lg, axis=1, keepdims=True)
    lse = jnp.log(jnp.sum(jnp.exp(lg - mm), axis=1, keepdims=True)) + mm
    out_ref[...] = lg - lse


def kernel(x, W1, b1, W2, b2, Wg, We, be, Wp, bp):
    N = x.shape[0]
    D = W1.shape[0]
    E = Wg.shape[1]
    NC = Wp.shape[1]
    EP = 128
    NCP = ((NC + 127) // 128) * 128
    NP = 2 * N + E * T
    NBLK = NP // T
    NBLKP = ((NBLK + LANES - 1) // LANES) * LANES
    TPW = N // NWORK
    G = N // BT

    xf = x.reshape(N, D)
    Wg_p = jnp.pad(Wg, ((0, 0), (0, EP - E)))
    Wp_p = jnp.pad(Wp, ((0, 0), (0, NCP - NC)))
    bp_p = jnp.pad(bp, (0, NCP - NC)).reshape(1, NCP)
    b1r = b1.reshape(1, D)
    b2r = b2.reshape(1, D)

    full = lambda *s: pl.BlockSpec(s, lambda i: (0,) * len(s))
    h, i1, i2, w1, w2, hc1, hc2 = pl.pallas_call(
        functools.partial(_k1_body, E),
        grid=(G,),
        in_specs=[
            pl.BlockSpec((BT, D), lambda i: (i, 0)),
            full(D, D), full(1, D), full(D, D), full(1, D), full(D, EP),
        ],
        out_specs=[
            pl.BlockSpec((BT, D), lambda i: (i, 0)),
            pl.BlockSpec((BT, 1), lambda i: (i, 0)),
            pl.BlockSpec((BT, 1), lambda i: (i, 0)),
            pl.BlockSpec((BT, 1), lambda i: (i, 0)),
            pl.BlockSpec((BT, 1), lambda i: (i, 0)),
            pl.BlockSpec((1, 2, EP), lambda i: (i, 0, 0)),
            pl.BlockSpec((1, 2, EP), lambda i: (i, 0, 0)),
        ],
        out_shape=[
            jax.ShapeDtypeStruct((N, D), jnp.bfloat16),
            jax.ShapeDtypeStruct((N, 1), jnp.int32),
            jax.ShapeDtypeStruct((N, 1), jnp.int32),
            jax.ShapeDtypeStruct((N, 1), jnp.float32),
            jax.ShapeDtypeStruct((N, 1), jnp.float32),
            jax.ShapeDtypeStruct((G, 2, EP), jnp.float32),
            jax.ShapeDtypeStruct((G, 2, EP), jnp.float32),
        ],
        compiler_params=pltpu.CompilerParams(
            dimension_semantics=("arbitrary",),
        ),
    )(xf, W1, b1r, W2, b2r, Wg_p)

    eid1 = i1.reshape(N)
    eid2 = i2.reshape(N)
    hc1 = hc1.reshape(NWORK, EP)
    hc2 = hc2.reshape(NWORK, EP)
    D2 = D // 2
    h32 = lax.bitcast_convert_type(h.reshape(N, D2, 2), jnp.int32)

    mesh = plsc.VectorSubcoreMesh(core_axis_name="c", subcore_axis_name="s",
                                  num_cores=2, num_subcores=16)
    k2 = pl.kernel(
        functools.partial(_k2_body, E, NP, NBLKP, TPW),
        out_type=[
            jax.ShapeDtypeStruct((NP, D2), jnp.int32),
            jax.ShapeDtypeStruct((N,), jnp.int32),
            jax.ShapeDtypeStruct((N,), jnp.int32),
            jax.ShapeDtypeStruct((NBLKP,), jnp.int32),
        ],
        mesh=mesh,
        scratch_types=[
            pltpu.VMEM((TPW,), jnp.int32),
            pltpu.VMEM((TPW,), jnp.int32),
            pltpu.VMEM((TPW,), jnp.int32),
            pltpu.VMEM((TPW,), jnp.int32),
            pltpu.VMEM((NWORK, EP), jnp.float32),
            pltpu.VMEM((NWORK, EP), jnp.float32),
            pltpu.VMEM((TPW, D2), jnp.int32),
            pltpu.VMEM((NBLKP,), jnp.int32),
            pltpu.SemaphoreType.DMA,
            pltpu.SemaphoreType.DMA,
        ],
    )
    hs32, pos1, pos2, blk = k2(h32, eid1, eid2, hc1, hc2)
    hs = lax.bitcast_convert_type(hs32, jnp.bfloat16).reshape(NP, D)

    y = pl.pallas_call(
        _k3_body,
        grid_spec=pltpu.PrefetchScalarGridSpec(
            num_scalar_prefetch=1,
            grid=(NBLK,),
            in_specs=[
                pl.BlockSpec((T, D), lambda b, s: (b, 0)),
                pl.BlockSpec((1, D, D), lambda b, s: (s[b], 0, 0)),
                pl.BlockSpec((1, 1, D), lambda b, s: (s[b], 0, 0)),
            ],
            out_specs=pl.BlockSpec((T, D), lambda b, s: (b, 0)),
        ),
        out_shape=jax.ShapeDtypeStruct((NP, D), jnp.bfloat16),
        compiler_params=pltpu.CompilerParams(
            dimension_semantics=("arbitrary",),
        ),
    )(blk, hs, We, be.reshape(E, 1, D))

    y32 = lax.bitcast_convert_type(y.reshape(NP, D2, 2), jnp.int32)
    k4 = pl.kernel(
        functools.partial(_k4_body, TPW),
        out_type=[
            jax.ShapeDtypeStruct((N, D2), jnp.int32),
            jax.ShapeDtypeStruct((N, D2), jnp.int32),
        ],
        mesh=mesh,
        scratch_types=[
            pltpu.VMEM((TPW,), jnp.int32),
            pltpu.VMEM((TPW,), jnp.int32),
            pltpu.VMEM((TPW, D2), jnp.int32),
            pltpu.SemaphoreType.DMA,
            pltpu.SemaphoreType.DMA,
        ],
    )
    y1g32, y2g32 = k4(y32, pos1, pos2)
    y1g = lax.bitcast_convert_type(y1g32, jnp.bfloat16).reshape(N, D)
    y2g = lax.bitcast_convert_type(y2g32, jnp.bfloat16).reshape(N, D)

    out = pl.pallas_call(
        functools.partial(_k5_body, NC),
        grid=(G,),
        in_specs=[
            pl.BlockSpec((BT, D), lambda i: (i, 0)),
            pl.BlockSpec((BT, D), lambda i: (i, 0)),
            pl.BlockSpec((BT, 1), lambda i: (i, 0)),
            pl.BlockSpec((BT, 1), lambda i: (i, 0)),
            full(D, NCP),
            full(1, NCP),
        ],
        out_specs=pl.BlockSpec((BT, NCP), lambda i: (i, 0)),
        out_shape=jax.ShapeDtypeStruct((N, NCP), jnp.float32),
        compiler_params=pltpu.CompilerParams(
            dimension_semantics=("arbitrary",),
        ),
    )(y1g, y2g, w1, w2, Wp_p, bp_p)
    return out[:, :NC]


# speedup vs baseline: 4.2247x; 4.2247x over previous
import functools

import jax
import jax.numpy as jnp
from jax import lax
from jax.experimental import pallas as pl
from jax.experimental.pallas import tpu as pltpu
from jax.experimental.pallas import tpu_sc as plsc

BT = 256
T = 256
NWORK = 32
LANES = 16


def _pack_rows(x):
    H = x.shape[1] // 2
    lo = lax.bitcast_convert_type(x[:, :H], jnp.uint32)
    hi = lax.bitcast_convert_type(x[:, H:], jnp.uint32)
    lo = lo + jnp.uint32(0x7FFF) + ((lo >> 16) & jnp.uint32(1))
    hi = hi + jnp.uint32(0x7FFF) + ((hi >> 16) & jnp.uint32(1))
    w = (hi & jnp.uint32(0xFFFF0000)) | (lo >> 16)
    return lax.bitcast_convert_type(w, jnp.int32)


def _unpack_rows(w):
    wu = lax.bitcast_convert_type(w, jnp.uint32)
    lo = lax.bitcast_convert_type(wu << 16, jnp.float32)
    hi = lax.bitcast_convert_type(wu & jnp.uint32(0xFFFF0000), jnp.float32)
    return jnp.concatenate([lo, hi], axis=1)


def _k1_body(E, x_ref, W1_ref, b1_ref, W2_ref, b2_ref, Wg_ref,
             h_ref, i1_ref, i2_ref, w1_ref, w2_ref, h1c_ref, h2c_ref):
    EP = Wg_ref.shape[1]
    bf = jnp.bfloat16
    x = x_ref[...]
    h = jnp.maximum(jnp.dot(x.astype(bf), W1_ref[...].astype(bf),
                            preferred_element_type=jnp.float32)
                    + b1_ref[...], 0.0)
    h = jnp.maximum(jnp.dot(h.astype(bf), W2_ref[...].astype(bf),
                            preferred_element_type=jnp.float32)
                    + b2_ref[...], 0.0)
    h_ref[...] = _pack_rows(h)
    logits = jnp.dot(h, Wg_ref[...], preferred_element_type=jnp.float32)
    col = lax.broadcasted_iota(jnp.int32, (BT, EP), 1)
    logits = jnp.where(col < E, logits, -1e30)
    m = jnp.max(logits, axis=1, keepdims=True)
    ex = jnp.exp(logits - m)
    gates = ex / jnp.sum(ex, axis=1, keepdims=True)
    m1 = jnp.max(gates, axis=1, keepdims=True)
    i1 = jnp.min(jnp.where(gates == m1, col, EP), axis=1, keepdims=True)
    g2 = jnp.where(col == i1, -1.0, gates)
    m2 = jnp.max(g2, axis=1, keepdims=True)
    i2 = jnp.min(jnp.where(g2 == m2, col, EP), axis=1, keepdims=True)
    denom = m1 + m2 + 1e-9
    i1_ref[...] = i1
    i2_ref[...] = i2
    w1_ref[...] = m1 / denom
    w2_ref[...] = m2 / denom
    oh1 = jnp.where(col == i1, 1.0, 0.0)
    oh2 = jnp.where(col == i2, 1.0, 0.0)
    half = BT // 2
    h1c_ref[...] = jnp.concatenate(
        [jnp.sum(oh1[:half], axis=0, keepdims=True),
         jnp.sum(oh1[half:], axis=0, keepdims=True)], axis=0)[None]
    h2c_ref[...] = jnp.concatenate(
        [jnp.sum(oh2[:half], axis=0, keepdims=True),
         jnp.sum(oh2[half:], axis=0, keepdims=True)], axis=0)[None]


def _gather16(v, idx):
    dnums = lax.GatherDimensionNumbers(
        offset_dims=(), collapsed_slice_dims=(0,), start_index_map=(0,))
    return lax.gather(v, idx[:, None], dnums, slice_sizes=(1,),
                      mode=lax.GatherScatterMode.PROMISE_IN_BOUNDS)


def _splat16(v, j):
    return _gather16(v, jnp.full((LANES,), j, jnp.int32))


def _prefix16(v, lane):
    for s in (1, 2, 4, 8):
        sh = _gather16(v, jnp.maximum(lane - s, 0))
        v = v + jnp.where(lane >= s, sh, jnp.zeros_like(v))
    return v


def _k2_body(E, NP, NBLKP, TPW,
             h_hbm, eid1_hbm, eid2_hbm, hc1_hbm, hc2_hbm,
             hs_hbm, pos1_hbm, pos2_hbm, blk_hbm,
             eid1_v, eid2_v, pos1_v, pos2_v,
             hc1_v, hc2_v, hbuf, blk_v,
             s1, s2):
    wid = lax.axis_index("s") * 2 + lax.axis_index("c")
    tbase = wid * TPW
    lane = lax.broadcasted_iota(jnp.int32, (LANES,), 0)

    pltpu.sync_copy(eid1_hbm.at[pl.ds(tbase, TPW)], eid1_v)
    pltpu.sync_copy(eid2_hbm.at[pl.ds(tbase, TPW)], eid2_v)
    pltpu.sync_copy(hc1_hbm, hc1_v)
    pltpu.sync_copy(hc2_hbm, hc2_v)

    zf = jnp.zeros((LANES,), jnp.float32)
    tot1 = zf
    tot2 = zf
    pre1 = zf
    pre2 = zf
    for w in range(NWORK):
        r1 = hc1_v[w, pl.ds(0, LANES)]
        r2 = hc2_v[w, pl.ds(0, LANES)]
        tot1 = tot1 + r1
        tot2 = tot2 + r2
        sel = w < wid
        pre1 = pre1 + jnp.where(sel, r1, zf)
        pre2 = pre2 + jnp.where(sel, r2, zf)
    tot = (tot1 + tot2).astype(jnp.int32)
    pc = ((tot + (T - 1)) >> 8) << 8
    pcs = [_splat16(pc, e) for e in range(E)]
    ps = [jnp.zeros((LANES,), jnp.int32)]
    for e in range(1, E):
        ps.append(ps[e - 1] + pcs[e - 1])
    tot1i = tot1.astype(jnp.int32)
    pre1i = pre1.astype(jnp.int32)
    pre2i = pre2.astype(jnp.int32)
    cur1 = [ps[e] + _splat16(pre1i, e) for e in range(E)]
    cur2 = [ps[e] + _splat16(tot1i, e) + _splat16(pre2i, e) for e in range(E)]
    ends = [ps[e] + pcs[e] for e in range(E)]

    @pl.when(wid == 0)
    def _():
        for c in range(NBLKP // LANES):
            bstart = (c * LANES + lane) * T
            acc = jnp.zeros((LANES,), jnp.int32)
            for e in range(E):
                acc = acc + jnp.where(ends[e] <= bstart, 1, 0)
            blk_v[pl.ds(c * LANES, LANES)] = jnp.minimum(acc, E - 1)
        pltpu.sync_copy(blk_v, blk_hbm)

    nch = TPW // LANES
    for eid_v, pos_v, cur in ((eid1_v, pos1_v, cur1), (eid2_v, pos2_v, cur2)):
        for c in range(nch):
            ev = eid_v[pl.ds(c * LANES, LANES)]
            posv = jnp.zeros((LANES,), jnp.int32)
            for e in range(E):
                msk = ev == e
                ones = jnp.where(msk, 1, 0)
                cs = _prefix16(ones, lane)
                posv = jnp.where(msk, cur[e] + cs - 1, posv)
                cur[e] = cur[e] + _splat16(cs, LANES - 1)
            pos_v[pl.ds(c * LANES, LANES)] = posv
    pltpu.sync_copy(pos1_v, pos1_hbm.at[pl.ds(tbase, TPW)])
    pltpu.sync_copy(pos2_v, pos2_hbm.at[pl.ds(tbase, TPW)])

    pltpu.sync_copy(h_hbm.at[pl.ds(tbase, TPW)], hbuf)
    c1 = pltpu.async_copy(hbuf, hs_hbm.at[pos1_v], s1)
    c2 = pltpu.async_copy(hbuf, hs_hbm.at[pos2_v], s2)
    c1.wait()
    c2.wait()


def _k3_body(blk_ref, hs_ref, We_ref, be_ref, y_ref):
    bf = jnp.bfloat16
    x = _unpack_rows(hs_ref[...]).astype(bf)
    y = jnp.dot(x, We_ref[0].astype(bf),
                preferred_element_type=jnp.float32) + be_ref[0]
    y_ref[...] = _pack_rows(y)


def _k4_body(TPW, y_hbm, pos1_hbm, pos2_hbm, y1_hbm, y2_hbm,
             pos1_v, pos2_v, yb, s1, s2):
    wid = lax.axis_index("s") * 2 + lax.axis_index("c")
    tbase = wid * TPW
    pltpu.sync_copy(pos1_hbm.at[pl.ds(tbase, TPW)], pos1_v)
    pltpu.sync_copy(pos2_hbm.at[pl.ds(tbase, TPW)], pos2_v)
    g1 = pltpu.async_copy(y_hbm.at[pos1_v], yb, s1)
    g1.wait()
    pltpu.sync_copy(yb, y1_hbm.at[pl.ds(tbase, TPW)])
    g2 = pltpu.async_copy(y_hbm.at[pos2_v], yb, s2)
    g2.wait()
    pltpu.sync_copy(yb, y2_hbm.at[pl.ds(tbase, TPW)])


def _k5_body(NC, y1_ref, y2_ref, w1_ref, w2_ref, Wp_ref, bp_ref, out_ref):
    bf = jnp.bfloat16
    NCP = Wp_ref.shape[1]
    hm = (w1_ref[...] * _unpack_rows(y1_ref[...])
          + w2_ref[...] * _unpack_rows(y2_ref[...]))
    lg = jnp.dot(hm.astype(bf), Wp_ref[...].astype(bf),
                 preferred_element_type=jnp.float32) + bp_ref[...]
    colc = lax.broadcasted_iota(jnp.int32, (BT, NCP), 1)
    lg = jnp.where(colc < NC, lg, -1e30)
    mm = jnp.max(lg, axis=1, keepdims=True)
    lse = jnp.log(jnp.sum(jnp.exp(lg - mm), axis=1, keepdims=True)) + mm
    out_ref[...] = lg - lse


def kernel(x, W1, b1, W2, b2, Wg, We, be, Wp, bp):
    N = x.shape[0]
    D = W1.shape[0]
    E = Wg.shape[1]
    NC = Wp.shape[1]
    EP = 128
    NCP = ((NC + 127) // 128) * 128
    NP = 2 * N + E * T
    NBLK = NP // T
    NBLKP = ((NBLK + LANES - 1) // LANES) * LANES
    TPW = N // NWORK
    G = N // BT

    D2 = D // 2
    xf = x.reshape(N, D)
    Wg_p = jnp.pad(Wg, ((0, 0), (0, EP - E)))
    Wp_p = jnp.pad(Wp, ((0, 0), (0, NCP - NC)))
    bp_p = jnp.pad(bp, (0, NCP - NC)).reshape(1, NCP)
    b1r = b1.reshape(1, D)
    b2r = b2.reshape(1, D)

    full = lambda *s: pl.BlockSpec(s, lambda i: (0,) * len(s))
    h, i1, i2, w1, w2, hc1, hc2 = pl.pallas_call(
        functools.partial(_k1_body, E),
        grid=(G,),
        in_specs=[
            pl.BlockSpec((BT, D), lambda i: (i, 0)),
            full(D, D), full(1, D), full(D, D), full(1, D), full(D, EP),
        ],
        out_specs=[
            pl.BlockSpec((BT, D2), lambda i: (i, 0)),
            pl.BlockSpec((BT, 1), lambda i: (i, 0)),
            pl.BlockSpec((BT, 1), lambda i: (i, 0)),
            pl.BlockSpec((BT, 1), lambda i: (i, 0)),
            pl.BlockSpec((BT, 1), lambda i: (i, 0)),
            pl.BlockSpec((1, 2, EP), lambda i: (i, 0, 0)),
            pl.BlockSpec((1, 2, EP), lambda i: (i, 0, 0)),
        ],
        out_shape=[
            jax.ShapeDtypeStruct((N, D2), jnp.int32),
            jax.ShapeDtypeStruct((N, 1), jnp.int32),
            jax.ShapeDtypeStruct((N, 1), jnp.int32),
            jax.ShapeDtypeStruct((N, 1), jnp.float32),
            jax.ShapeDtypeStruct((N, 1), jnp.float32),
            jax.ShapeDtypeStruct((G, 2, EP), jnp.float32),
            jax.ShapeDtypeStruct((G, 2, EP), jnp.float32),
        ],
        compiler_params=pltpu.CompilerParams(
            dimension_semantics=("arbitrary",),
        ),
    )(xf, W1, b1r, W2, b2r, Wg_p)

    eid1 = i1.reshape(N)
    eid2 = i2.reshape(N)
    hc1 = hc1.reshape(NWORK, EP)
    hc2 = hc2.reshape(NWORK, EP)

    mesh = plsc.VectorSubcoreMesh(core_axis_name="c", subcore_axis_name="s",
                                  num_cores=2, num_subcores=16)
    k2 = pl.kernel(
        functools.partial(_k2_body, E, NP, NBLKP, TPW),
        out_type=[
            jax.ShapeDtypeStruct((NP, D2), jnp.int32),
            jax.ShapeDtypeStruct((N,), jnp.int32),
            jax.ShapeDtypeStruct((N,), jnp.int32),
            jax.ShapeDtypeStruct((NBLKP,), jnp.int32),
        ],
        mesh=mesh,
        scratch_types=[
            pltpu.VMEM((TPW,), jnp.int32),
            pltpu.VMEM((TPW,), jnp.int32),
            pltpu.VMEM((TPW,), jnp.int32),
            pltpu.VMEM((TPW,), jnp.int32),
            pltpu.VMEM((NWORK, EP), jnp.float32),
            pltpu.VMEM((NWORK, EP), jnp.float32),
            pltpu.VMEM((TPW, D2), jnp.int32),
            pltpu.VMEM((NBLKP,), jnp.int32),
            pltpu.SemaphoreType.DMA,
            pltpu.SemaphoreType.DMA,
        ],
    )
    hs, pos1, pos2, blk = k2(h, eid1, eid2, hc1, hc2)

    y = pl.pallas_call(
        _k3_body,
        grid_spec=pltpu.PrefetchScalarGridSpec(
            num_scalar_prefetch=1,
            grid=(NBLK,),
            in_specs=[
                pl.BlockSpec((T, D2), lambda b, s: (b, 0)),
                pl.BlockSpec((1, D, D), lambda b, s: (s[b], 0, 0)),
                pl.BlockSpec((1, 1, D), lambda b, s: (s[b], 0, 0)),
            ],
            out_specs=pl.BlockSpec((T, D2), lambda b, s: (b, 0)),
        ),
        out_shape=jax.ShapeDtypeStruct((NP, D2), jnp.int32),
        compiler_params=pltpu.CompilerParams(
            dimension_semantics=("arbitrary",),
        ),
    )(blk, hs, We, be.reshape(E, 1, D))

    k4 = pl.kernel(
        functools.partial(_k4_body, TPW),
        out_type=[
            jax.ShapeDtypeStruct((N, D2), jnp.int32),
            jax.ShapeDtypeStruct((N, D2), jnp.int32),
        ],
        mesh=mesh,
        scratch_types=[
            pltpu.VMEM((TPW,), jnp.int32),
            pltpu.VMEM((TPW,), jnp.int32),
            pltpu.VMEM((TPW, D2), jnp.int32),
            pltpu.SemaphoreType.DMA,
            pltpu.SemaphoreType.DMA,
        ],
    )
    y1g, y2g = k4(y, pos1, pos2)

    out = pl.pallas_call(
        functools.partial(_k5_body, NC),
        grid=(G,),
        in_specs=[
            pl.BlockSpec((BT, D2), lambda i: (i, 0)),
            pl.BlockSpec((BT, D2), lambda i: (i, 0)),
            pl.BlockSpec((BT, 1), lambda i: (i, 0)),
            pl.BlockSpec((BT, 1), lambda i: (i, 0)),
            full(D, NCP),
            full(1, NCP),
        ],
        out_specs=pl.BlockSpec((BT, NCP), lambda i: (i, 0)),
        out_shape=jax.ShapeDtypeStruct((N, NCP), jnp.float32),
        compiler_params=pltpu.CompilerParams(
            dimension_semantics=("arbitrary",),
        ),
    )(y1g, y2g, w1, w2, Wp_p, bp_p)
    return out[:, :NC]
